# baseline (speedup 1.0000x reference)
import functools

import jax
import jax.numpy as jnp
from jax import lax
from jax.experimental import pallas as pl
from jax.experimental.pallas import tpu as pltpu
from jax.experimental.pallas import tpu_sc as plsc

N = 10000
NP = 10240
E = 160000
CH = 128
NCH_TOTAL = 1280
EP = NCH_TOTAL * CH
NTILES = 32
TCH = NCH_TOTAL // NTILES
NCH_RAW = E // CH
RPT = NP // 16
TAIL = N - 15 * RPT
DEG_F = 16

_MESH = dict(core_axis_name="c", subcore_axis_name="s", num_cores=2,
             num_subcores=16)
_SC_PARAMS = pltpu.CompilerParams(use_tc_tiling_on_sc=False)


def _edge_scatter_kernel(feat, cw=256):
    ng = EP // cw // NTILES

    @functools.partial(
        pl.kernel,
        out_type=jax.ShapeDtypeStruct((2, NP, feat), jnp.float32),
        mesh=plsc.VectorSubcoreMesh(**_MESH),
        compiler_params=_SC_PARAMS,
        scratch_types=[
            pltpu.VMEM((EP // cw // NTILES, cw), jnp.int32),
            pltpu.VMEM((EP // cw // NTILES, cw), jnp.int32),
            pltpu.VMEM((cw, feat), jnp.float32),
            pltpu.VMEM((cw, feat), jnp.float32),
            pltpu.VMEM((cw, feat), jnp.float32),
            pltpu.VMEM((cw, feat), jnp.float32),
            pltpu.VMEM_SHARED((NP, feat), jnp.float32),
            pltpu.VMEM_SHARED((NP, feat), jnp.float32),
            pltpu.SemaphoreType.DMA,
            pltpu.SemaphoreType.DMA,
            pltpu.SemaphoreType.DMA,
            pltpu.SemaphoreType.DMA,
            pltpu.SemaphoreType.DMA,
            pltpu.SemaphoreType.DMA,
            pltpu.SemaphoreType.DMA,
            pltpu.SemaphoreType.DMA,
        ],
    )
    def sc_kernel(src_hbm, dst_hbm, table_hbm, zeros_hbm, out_hbm,
                  srcv, dstv, m0, m1, m2, m3, tab, acc,
                  g0, g1, g2, g3, c0, c1, c2, c3):
        bufs = (m0, m1, m2, m3)
        gsem = (g0, g1, g2, g3)
        csem = (c0, c1, c2, c3)
        cid = lax.axis_index("c")
        sid = lax.axis_index("s")
        wid = sid * 2 + cid
        rows = pl.ds(sid * RPT, RPT)

        def gather(j, b):
            return pltpu.async_copy(tab.at[srcv.at[j]], bufs[b], gsem[b])

        def gather_wait(b):
            pltpu.make_async_copy(tab.at[srcv.at[0]], bufs[b], gsem[b]).wait()

        def scatter(j, b):
            return pltpu.async_copy(bufs[b], acc.at[dstv.at[j]], csem[b],
                                    add=True)

        def scatter_wait(b):
            pltpu.make_async_copy(bufs[b], acc.at[dstv.at[0]], csem[b]).wait()

        @pl.when(sid < 15)
        def _():
            pltpu.sync_copy(table_hbm.at[rows], tab.at[rows])

        @pl.when(sid == 15)
        def _():
            tail = pl.ds(15 * RPT, TAIL)
            pltpu.sync_copy(table_hbm.at[tail], tab.at[tail])

        pltpu.sync_copy(zeros_hbm, acc.at[rows])
        pltpu.sync_copy(src_hbm.at[pl.ds(wid * ng, ng)], srcv)
        pltpu.sync_copy(dst_hbm.at[pl.ds(wid * ng, ng)], dstv)
        plsc.subcore_barrier()

        gather(0, 0)
        gather(1, 1)
        gather(2, 2)
        gather_wait(0)
        gather(3, 3)
        scatter(0, 0)
        for j in (1, 2, 3):
            gather_wait(j)
            scatter_wait(j - 1)
            gather(j + 3, (j + 3) % 4)
            scatter(j, j)

        def body(q, carry):
            for r in range(4):
                j = 4 + 4 * q + r
                nb = (r + 3) % 4
                gather_wait(r)
                scatter_wait(nb)
                jn = jnp.minimum(j + 3, ng - 1)
                gather(jn, nb)
                scatter(j, r)
            return carry

        lax.fori_loop(0, (ng - 4) // 4, body, 0)
        scatter_wait(3)
        for b in range(3):
            gather_wait(b)
        plsc.subcore_barrier()
        pltpu.sync_copy(acc.at[rows], out_hbm.at[cid, rows])

    return sc_kernel


def _degree_kernel():
    @functools.partial(
        pl.kernel,
        out_type=jax.ShapeDtypeStruct((2, NP, DEG_F), jnp.float32),
        mesh=plsc.VectorSubcoreMesh(**_MESH),
        compiler_params=_SC_PARAMS,
        scratch_types=[
            pltpu.VMEM((TCH, CH), jnp.int32),
            pltpu.VMEM((CH, DEG_F), jnp.float32),
            pltpu.VMEM((RPT, DEG_F), jnp.float32),
            pltpu.VMEM_SHARED((NP, DEG_F), jnp.float32),
            pltpu.SemaphoreType.DMA,
        ],
    )
    def deg_kernel(dst_hbm, ones_hbm, zeros_hbm, out_hbm,
                   dstv, onesv, stage, acc, sem):
        cid = lax.axis_index("c")
        sid = lax.axis_index("s")
        wid = sid * 2 + cid
        rows = pl.ds(sid * RPT, RPT)
        base = wid * NCH_RAW // NTILES
        tch = (wid + 1) * NCH_RAW // NTILES - base
        pltpu.sync_copy(zeros_hbm, stage)
        pltpu.sync_copy(stage, acc.at[rows])
        pltpu.sync_copy(ones_hbm, onesv)
        pltpu.sync_copy(dst_hbm.at[pl.ds(base, TCH)], dstv)
        plsc.subcore_barrier()

        def fire(j, carry):
            pltpu.async_copy(onesv, acc.at[dstv.at[j]], sem, add=True)
            return carry

        lax.fori_loop(0, tch, fire, 0)

        def drain(j, carry):
            pltpu.make_async_copy(onesv, acc.at[dstv.at[0]], sem).wait()
            return carry

        lax.fori_loop(0, tch, drain, 0)
        plsc.subcore_barrier()
        pltpu.sync_copy(acc.at[rows], out_hbm.at[cid, rows])

    return deg_kernel


_R = 1000


def _tc1_body(deg_ref, x_ref, w_ref, y_ref, dinv_ref):
    deg = deg_ref[0, :, :1] + deg_ref[1, :, :1] + 1.0
    dinv = lax.rsqrt(deg)
    h = jnp.dot(x_ref[...], w_ref[...], preferred_element_type=jnp.float32)
    y_ref[...] = h * dinv
    dinv_ref[...] = dinv


def _tc_mid_body(s_ref, y_ref, dinv_ref, b_ref, w_ref, o_ref):
    dinv = dinv_ref[...]
    t = (s_ref[0] + s_ref[1] + y_ref[...]) * dinv + b_ref[...][None, :]
    z = jnp.maximum(t, 0.0)
    h = jnp.dot(z, w_ref[...], preferred_element_type=jnp.float32)
    o_ref[...] = h * dinv


def _tc_last_body(s_ref, y_ref, dinv_ref, b_ref, w_ref, b4_ref, o_ref):
    dinv = dinv_ref[...]
    t = (s_ref[0] + s_ref[1] + y_ref[...]) * dinv + b_ref[...][None, :]
    z = jnp.maximum(t, 0.0)
    o_ref[...] = (jnp.dot(z, w_ref[...], preferred_element_type=jnp.float32)
                  + b4_ref[...][None, :])


def _row_spec(feat):
    return pl.BlockSpec((_R, feat), lambda i: (i, 0))


def _vec_spec():
    return pl.BlockSpec((_R, 1), lambda i: (i, 0))


def _pair_spec(feat):
    return pl.BlockSpec((2, _R, feat), lambda i: (0, i, 0))


def _full_spec(*shape):
    n = len(shape)
    return pl.BlockSpec(shape, lambda i: (0,) * n)


def _tc1(deg_p, x, w1):
    return pl.pallas_call(
        _tc1_body,
        grid=(N // _R,),
        in_specs=[pl.BlockSpec((2, _R, DEG_F), lambda i: (0, i, 0)),
                  _row_spec(256), _full_spec(256, 64)],
        out_specs=[_row_spec(64), _vec_spec()],
        out_shape=[jax.ShapeDtypeStruct((N, 64), jnp.float32),
                   jax.ShapeDtypeStruct((N, 1), jnp.float32)],
    )(deg_p, x, w1)


def _tc_mid(s, y, dinv, b, w, f_in, f_out):
    return pl.pallas_call(
        _tc_mid_body,
        grid=(N // _R,),
        in_specs=[_pair_spec(f_in), _row_spec(f_in), _vec_spec(),
                  _full_spec(f_in), _full_spec(f_in, f_out)],
        out_specs=_row_spec(f_out),
        out_shape=jax.ShapeDtypeStruct((N, f_out), jnp.float32),
    )(s, y, dinv, b, w)


def _tc_last(s, y, dinv, b3, w4, b4):
    return pl.pallas_call(
        _tc_last_body,
        grid=(N // _R,),
        in_specs=[_pair_spec(16), _row_spec(16), _vec_spec(),
                  _full_spec(16), _full_spec(16, 2), _full_spec(2)],
        out_specs=_row_spec(2),
        out_shape=jax.ShapeDtypeStruct((N, 2), jnp.float32),
    )(s, y, dinv, b3, w4, b4)


def kernel(x, edge_index, W1, b1, W2, b2, W3, b3, W4, b4):
    src = edge_index[0].astype(jnp.int32)
    dst = edge_index[1].astype(jnp.int32)
    pad = EP - E
    src = jnp.concatenate([src, jnp.zeros((pad,), jnp.int32)])
    dst = jnp.concatenate([dst, jnp.full((pad,), NP - 1, jnp.int32)])
    src128 = src.reshape(EP // 128, 128)
    dst128 = dst.reshape(EP // 128, 128)
    src256 = src.reshape(EP // 256, 256)
    dst256 = dst.reshape(EP // 256, 256)

    ones_deg = jnp.ones((CH, DEG_F), jnp.float32)
    zeros_deg = jnp.zeros((RPT, DEG_F), jnp.float32)

    dst_raw = edge_index[1].astype(jnp.int32).reshape(NCH_RAW, CH)
    deg_p = _degree_kernel()(dst_raw, ones_deg, zeros_deg)

    y1, dinv = _tc1(deg_p, x, W1)

    s1 = _edge_scatter_kernel(64, 128)(src128, dst128, y1, jnp.zeros((RPT, 64), jnp.float32))
    y2 = _tc_mid(s1, y1, dinv, b1, W2, 64, 32)

    s2 = _edge_scatter_kernel(32, 256)(src256, dst256, y2, jnp.zeros((RPT, 32), jnp.float32))
    y3 = _tc_mid(s2, y2, dinv, b2, W3, 32, 16)

    s3 = _edge_scatter_kernel(16, 256)(src256, dst256, y3, jnp.zeros((RPT, 16), jnp.float32))
    return _tc_last(s3, y3, dinv, b3, W4, b4)


# speedup vs baseline: 1.1895x; 1.1895x over previous
import functools

import jax
import jax.numpy as jnp
from jax import lax
from jax.experimental import pallas as pl
from jax.experimental.pallas import tpu as pltpu
from jax.experimental.pallas import tpu_sc as plsc

N = 10000
NP = 10240
E = 160000
CH = 128
NCH_TOTAL = 1280
EP = NCH_TOTAL * CH
NTILES = 32
TCH = NCH_TOTAL // NTILES
NCH_RAW = E // CH
RPT = NP // 16
TAIL = N - 15 * RPT
DEG_F = 16

_MESH = dict(core_axis_name="c", subcore_axis_name="s", num_cores=2,
             num_subcores=16)
_SC_PARAMS = pltpu.CompilerParams(use_tc_tiling_on_sc=False)


def _edge_scatter_kernel(feat, cw=256):
    ng = EP // cw // NTILES

    @functools.partial(
        pl.kernel,
        out_type=jax.ShapeDtypeStruct((2, NP, feat), jnp.bfloat16),
        mesh=plsc.VectorSubcoreMesh(**_MESH),
        compiler_params=_SC_PARAMS,
        scratch_types=[
            pltpu.VMEM((EP // cw // NTILES, cw), jnp.int32),
            pltpu.VMEM((EP // cw // NTILES, cw), jnp.int32),
            pltpu.VMEM((cw, feat), jnp.bfloat16),
            pltpu.VMEM((cw, feat), jnp.bfloat16),
            pltpu.VMEM((cw, feat), jnp.bfloat16),
            pltpu.VMEM((cw, feat), jnp.bfloat16),
            pltpu.VMEM_SHARED((NP, feat), jnp.bfloat16),
            pltpu.VMEM_SHARED((NP, feat), jnp.bfloat16),
            pltpu.SemaphoreType.DMA,
            pltpu.SemaphoreType.DMA,
            pltpu.SemaphoreType.DMA,
            pltpu.SemaphoreType.DMA,
            pltpu.SemaphoreType.DMA,
            pltpu.SemaphoreType.DMA,
            pltpu.SemaphoreType.DMA,
            pltpu.SemaphoreType.DMA,
        ],
    )
    def sc_kernel(src_hbm, dst_hbm, table_hbm, zeros_hbm, out_hbm,
                  srcv, dstv, m0, m1, m2, m3, tab, acc,
                  g0, g1, g2, g3, c0, c1, c2, c3):
        bufs = (m0, m1, m2, m3)
        gsem = (g0, g1, g2, g3)
        csem = (c0, c1, c2, c3)
        cid = lax.axis_index("c")
        sid = lax.axis_index("s")
        wid = sid * 2 + cid
        rows = pl.ds(sid * RPT, RPT)

        def gather(j, b):
            return pltpu.async_copy(tab.at[srcv.at[j]], bufs[b], gsem[b])

        def gather_wait(b):
            pltpu.make_async_copy(tab.at[srcv.at[0]], bufs[b], gsem[b]).wait()

        def scatter(j, b):
            return pltpu.async_copy(bufs[b], acc.at[dstv.at[j]], csem[b],
                                    add=True)

        def scatter_wait(b):
            pltpu.make_async_copy(bufs[b], acc.at[dstv.at[0]], csem[b]).wait()

        @pl.when(sid < 15)
        def _():
            pltpu.sync_copy(table_hbm.at[rows], tab.at[rows])

        @pl.when(sid == 15)
        def _():
            tail = pl.ds(15 * RPT, TAIL)
            pltpu.sync_copy(table_hbm.at[tail], tab.at[tail])

        pltpu.sync_copy(zeros_hbm, acc.at[rows])
        pltpu.sync_copy(src_hbm.at[pl.ds(wid * ng, ng)], srcv)
        pltpu.sync_copy(dst_hbm.at[pl.ds(wid * ng, ng)], dstv)
        plsc.subcore_barrier()

        gather(0, 0)
        gather(1, 1)
        gather(2, 2)
        gather_wait(0)
        gather(3, 3)
        scatter(0, 0)
        for j in (1, 2, 3):
            gather_wait(j)
            scatter_wait(j - 1)
            gather(j + 3, (j + 3) % 4)
            scatter(j, j)

        def body(q, carry):
            for r in range(4):
                j = 4 + 4 * q + r
                nb = (r + 3) % 4
                gather_wait(r)
                scatter_wait(nb)
                jn = jnp.minimum(j + 3, ng - 1)
                gather(jn, nb)
                scatter(j, r)
            return carry

        lax.fori_loop(0, (ng - 4) // 4, body, 0)
        scatter_wait(3)
        for b in range(3):
            gather_wait(b)
        plsc.subcore_barrier()
        pltpu.sync_copy(acc.at[rows], out_hbm.at[cid, rows])

    return sc_kernel


def _degree_kernel():
    @functools.partial(
        pl.kernel,
        out_type=jax.ShapeDtypeStruct((2, NP, DEG_F), jnp.float32),
        mesh=plsc.VectorSubcoreMesh(**_MESH),
        compiler_params=_SC_PARAMS,
        scratch_types=[
            pltpu.VMEM((TCH, CH), jnp.int32),
            pltpu.VMEM((CH, DEG_F), jnp.float32),
            pltpu.VMEM((RPT, DEG_F), jnp.float32),
            pltpu.VMEM_SHARED((NP, DEG_F), jnp.float32),
            pltpu.SemaphoreType.DMA,
        ],
    )
    def deg_kernel(dst_hbm, ones_hbm, zeros_hbm, out_hbm,
                   dstv, onesv, stage, acc, sem):
        cid = lax.axis_index("c")
        sid = lax.axis_index("s")
        wid = sid * 2 + cid
        rows = pl.ds(sid * RPT, RPT)
        base = wid * NCH_RAW // NTILES
        tch = (wid + 1) * NCH_RAW // NTILES - base
        pltpu.sync_copy(zeros_hbm, stage)
        pltpu.sync_copy(stage, acc.at[rows])
        pltpu.sync_copy(ones_hbm, onesv)
        pltpu.sync_copy(dst_hbm.at[pl.ds(base, TCH)], dstv)
        plsc.subcore_barrier()

        def fire(j, carry):
            pltpu.async_copy(onesv, acc.at[dstv.at[j]], sem, add=True)
            return carry

        lax.fori_loop(0, tch, fire, 0)

        def drain(j, carry):
            pltpu.make_async_copy(onesv, acc.at[dstv.at[0]], sem).wait()
            return carry

        lax.fori_loop(0, tch, drain, 0)
        plsc.subcore_barrier()
        pltpu.sync_copy(acc.at[rows], out_hbm.at[cid, rows])

    return deg_kernel


_R = 1000


def _tc1_body(deg_ref, x_ref, w_ref, y_ref, dinv_ref):
    deg = deg_ref[0, :, :1] + deg_ref[1, :, :1] + 1.0
    dinv = lax.rsqrt(deg)
    h = jnp.dot(x_ref[...], w_ref[...], preferred_element_type=jnp.float32)
    y_ref[...] = (h * dinv).astype(jnp.bfloat16)
    dinv_ref[...] = dinv


def _tc_mid_body(s_ref, y_ref, dinv_ref, b_ref, w_ref, o_ref):
    dinv = dinv_ref[...]
    s = (s_ref[0].astype(jnp.float32) + s_ref[1].astype(jnp.float32)
         + y_ref[...].astype(jnp.float32))
    t = s * dinv + b_ref[...][None, :]
    z = jnp.maximum(t, 0.0)
    h = jnp.dot(z, w_ref[...], preferred_element_type=jnp.float32)
    o_ref[...] = (h * dinv).astype(jnp.bfloat16)


def _tc_last_body(s_ref, y_ref, dinv_ref, b_ref, w_ref, b4_ref, o_ref):
    dinv = dinv_ref[...]
    s = (s_ref[0].astype(jnp.float32) + s_ref[1].astype(jnp.float32)
         + y_ref[...].astype(jnp.float32))
    t = s * dinv + b_ref[...][None, :]
    z = jnp.maximum(t, 0.0)
    o_ref[...] = (jnp.dot(z, w_ref[...], preferred_element_type=jnp.float32)
                  + b4_ref[...][None, :])


def _row_spec(feat):
    return pl.BlockSpec((_R, feat), lambda i: (i, 0))


def _vec_spec():
    return pl.BlockSpec((_R, 1), lambda i: (i, 0))


def _pair_spec(feat):
    return pl.BlockSpec((2, _R, feat), lambda i: (0, i, 0))


def _full_spec(*shape):
    n = len(shape)
    return pl.BlockSpec(shape, lambda i: (0,) * n)


def _tc1(deg_p, x, w1):
    return pl.pallas_call(
        _tc1_body,
        grid=(N // _R,),
        in_specs=[pl.BlockSpec((2, _R, DEG_F), lambda i: (0, i, 0)),
                  _row_spec(256), _full_spec(256, 64)],
        out_specs=[_row_spec(64), _vec_spec()],
        out_shape=[jax.ShapeDtypeStruct((N, 64), jnp.bfloat16),
                   jax.ShapeDtypeStruct((N, 1), jnp.float32)],
    )(deg_p, x, w1)


def _tc_mid(s, y, dinv, b, w, f_in, f_out):
    return pl.pallas_call(
        _tc_mid_body,
        grid=(N // _R,),
        in_specs=[_pair_spec(f_in), _row_spec(f_in), _vec_spec(),
                  _full_spec(f_in), _full_spec(f_in, f_out)],
        out_specs=_row_spec(f_out),
        out_shape=jax.ShapeDtypeStruct((N, f_out), jnp.bfloat16),
    )(s, y, dinv, b, w)


def _tc_last(s, y, dinv, b3, w4, b4):
    return pl.pallas_call(
        _tc_last_body,
        grid=(N // _R,),
        in_specs=[_pair_spec(16), _row_spec(16), _vec_spec(),
                  _full_spec(16), _full_spec(16, 2), _full_spec(2)],
        out_specs=_row_spec(2),
        out_shape=jax.ShapeDtypeStruct((N, 2), jnp.float32),
    )(s, y, dinv, b3, w4, b4)


def kernel(x, edge_index, W1, b1, W2, b2, W3, b3, W4, b4):
    src = edge_index[0].astype(jnp.int32)
    dst = edge_index[1].astype(jnp.int32)
    pad = EP - E
    src = jnp.concatenate([src, jnp.zeros((pad,), jnp.int32)])
    dst = jnp.concatenate([dst, jnp.full((pad,), NP - 1, jnp.int32)])
    src128 = src.reshape(EP // 128, 128)
    dst128 = dst.reshape(EP // 128, 128)

    ones_deg = jnp.ones((CH, DEG_F), jnp.float32)
    zeros_deg = jnp.zeros((RPT, DEG_F), jnp.float32)

    dst_raw = edge_index[1].astype(jnp.int32).reshape(NCH_RAW, CH)
    deg_p = _degree_kernel()(dst_raw, ones_deg, zeros_deg)

    y1, dinv = _tc1(deg_p, x, W1)

    s1 = _edge_scatter_kernel(64, 128)(src128, dst128, y1, jnp.zeros((RPT, 64), jnp.bfloat16))
    y2 = _tc_mid(s1, y1, dinv, b1, W2, 64, 32)

    s2 = _edge_scatter_kernel(32, 128)(src128, dst128, y2, jnp.zeros((RPT, 32), jnp.bfloat16))
    y3 = _tc_mid(s2, y2, dinv, b2, W3, 32, 16)

    s3 = _edge_scatter_kernel(16, 128)(src128, dst128, y3, jnp.zeros((RPT, 16), jnp.bfloat16))
    return _tc_last(s3, y3, dinv, b3, W4, b4)
